# baseline (speedup 1.0000x reference)
import jax
import jax.numpy as jnp
from jax.experimental import pallas as pl

B = 4096
TWO_N = 128
N = 64
OUT_W = TWO_N * N + N + 2
ROWS = 128


def _body(card_ref, seen_ref, flip_ref, valid_ref, t_ref, out3_ref):
    out_ref = out3_ref.at[:, 0, :]
    R = card_ref.shape[0]
    lane = jax.lax.broadcasted_iota(jnp.int32, (R, 128), 1)
    mod64 = jnp.bitwise_and(lane, 63)
    hi = lane >= 64

    card = card_ref[...]
    seen = seen_ref[...]
    cardm = jnp.where(seen, card, 64)

    for i in range(N):
        c0 = cardm[:, 2 * i : 2 * i + 1]
        c1 = cardm[:, 2 * i + 1 : 2 * i + 2]
        csel = jnp.where(hi, c1, c0)
        out_ref[:, 128 * i : 128 * (i + 1)] = jnp.where(
            csel == mod64, 1.0, 0.0
        )

    f = flip_ref[...]
    fc = jnp.sum(jnp.where(lane == f, card, 0), axis=1, keepdims=True)
    valid = valid_ref[...]
    par = jnp.bitwise_and(t_ref[...], 1)
    flip_val = jnp.where(lane == fc, valid, 0.0)
    par_val = jnp.where((lane - 64) == par, 1.0, 0.0)
    tail = jnp.where(lane < 64, flip_val, par_val)
    out_ref[:, TWO_N * N : OUT_W] = tail[:, : N + 2]


def kernel(card, seen_mask, flipped, flipped_valid, t, W):
    del W
    card = card.astype(jnp.int32)
    seen = seen_mask
    flip = flipped.astype(jnp.int32).reshape(B, 1)
    valid = flipped_valid.astype(jnp.float32).reshape(B, 1)
    t32 = t.astype(jnp.int32).reshape(B, 1)

    grid = (B // ROWS,)
    out = pl.pallas_call(
        _body,
        grid=grid,
        in_specs=[
            pl.BlockSpec((ROWS, TWO_N), lambda i: (i, 0)),
            pl.BlockSpec((ROWS, TWO_N), lambda i: (i, 0)),
            pl.BlockSpec((ROWS, 1), lambda i: (i, 0)),
            pl.BlockSpec((ROWS, 1), lambda i: (i, 0)),
            pl.BlockSpec((ROWS, 1), lambda i: (i, 0)),
        ],
        out_specs=pl.BlockSpec((ROWS, 1, OUT_W), lambda i: (i, 0, 0)),
        out_shape=jax.ShapeDtypeStruct((B, 1, OUT_W), jnp.float32),
    )(card, seen, flip, valid, t32)
    return out


# speedup vs baseline: 1.0006x; 1.0006x over previous
import jax
import jax.numpy as jnp
from jax.experimental import pallas as pl

B = 4096
TWO_N = 128
N = 64
OUT_W = TWO_N * N + N + 2
ROWS = 128


def _body(card_ref, seen_ref, flip_ref, valid_ref, t_ref, out_ref):
    R = card_ref.shape[0]
    lane = jax.lax.broadcasted_iota(jnp.int32, (R, 128), 1)
    mod64 = jnp.bitwise_and(lane, 63)
    hi = lane >= 64

    card = card_ref[...]
    seen = seen_ref[...]
    cardm = jnp.where(seen, card, 64)

    for i in range(N):
        c0 = cardm[:, 2 * i : 2 * i + 1]
        c1 = cardm[:, 2 * i + 1 : 2 * i + 2]
        csel = jnp.where(hi, c1, c0)
        out_ref[:, 128 * i : 128 * (i + 1)] = jnp.where(
            csel == mod64, 1.0, 0.0
        )

    f = flip_ref[...]
    fc = jnp.sum(jnp.where(lane == f, card, 0), axis=1, keepdims=True)
    valid = valid_ref[...]
    par = jnp.bitwise_and(t_ref[...], 1)
    flip_val = jnp.where(lane == fc, valid, 0.0)
    par_val = jnp.where((lane - 64) == par, 1.0, 0.0)
    tail = jnp.where(lane < 64, flip_val, par_val)
    out_ref[:, TWO_N * N : OUT_W] = tail[:, : N + 2]


def kernel(card, seen_mask, flipped, flipped_valid, t, W):
    del W
    card = card.astype(jnp.int32)
    seen = seen_mask
    flip = flipped.astype(jnp.int32).reshape(B, 1)
    valid = flipped_valid.astype(jnp.float32).reshape(B, 1)
    t32 = t.astype(jnp.int32).reshape(B, 1)

    grid = (B // ROWS,)
    out = pl.pallas_call(
        _body,
        grid=grid,
        in_specs=[
            pl.BlockSpec((ROWS, TWO_N), lambda i: (i, 0)),
            pl.BlockSpec((ROWS, TWO_N), lambda i: (i, 0)),
            pl.BlockSpec((ROWS, 1), lambda i: (i, 0)),
            pl.BlockSpec((ROWS, 1), lambda i: (i, 0)),
            pl.BlockSpec((ROWS, 1), lambda i: (i, 0)),
        ],
        out_specs=pl.BlockSpec((ROWS, None, OUT_W), lambda i: (i, 0, 0)),
        out_shape=jax.ShapeDtypeStruct((B, 1, OUT_W), jnp.float32),
    )(card, seen, flip, valid, t32)
    return out


# speedup vs baseline: 2.7612x; 2.7594x over previous
import jax
import jax.numpy as jnp
from jax.experimental import pallas as pl

B = 4096
TWO_N = 128
N = 64
OUT_W = TWO_N * N + N + 2
ROWS = 256


def _body(card_ref, seen_ref, flip_ref, valid_ref, t_ref, out_ref):
    R = card_ref.shape[0]
    lane = jax.lax.broadcasted_iota(jnp.int32, (R, 128), 1)
    mod64 = jnp.bitwise_and(lane, 63)
    hi = lane >= 64

    card = card_ref[...]
    seen = seen_ref[...]
    cardm = jnp.where(seen, card, 64)

    for i in range(N):
        c0 = cardm[:, 2 * i : 2 * i + 1]
        c1 = cardm[:, 2 * i + 1 : 2 * i + 2]
        csel = jnp.where(hi, c1, c0)
        out_ref[:, 128 * i : 128 * (i + 1)] = jnp.where(
            csel == mod64, 1.0, 0.0
        )

    f = flip_ref[...]
    fc = jnp.sum(jnp.where(lane == f, card, 0), axis=1, keepdims=True)
    valid = valid_ref[...]
    par = jnp.bitwise_and(t_ref[...], 1)
    flip_val = jnp.where(lane == fc, valid, 0.0)
    par_val = jnp.where((lane - 64) == par, 1.0, 0.0)
    tail = jnp.where(lane < 64, flip_val, par_val)
    out_ref[:, TWO_N * N : OUT_W] = tail[:, : N + 2]


def kernel(card, seen_mask, flipped, flipped_valid, t, W):
    del W
    card = card.astype(jnp.int32)
    seen = seen_mask
    flip = flipped.astype(jnp.int32).reshape(B, 1)
    valid = flipped_valid.astype(jnp.float32).reshape(B, 1)
    t32 = t.astype(jnp.int32).reshape(B, 1)

    grid = (B // ROWS,)
    out = pl.pallas_call(
        _body,
        grid=grid,
        in_specs=[
            pl.BlockSpec((ROWS, TWO_N), lambda i: (i, 0)),
            pl.BlockSpec((ROWS, TWO_N), lambda i: (i, 0)),
            pl.BlockSpec((ROWS, 1), lambda i: (i, 0)),
            pl.BlockSpec((ROWS, 1), lambda i: (i, 0)),
            pl.BlockSpec((ROWS, 1), lambda i: (i, 0)),
        ],
        out_specs=pl.BlockSpec((ROWS, OUT_W), lambda i: (i, 0)),
        out_shape=jax.ShapeDtypeStruct((B, OUT_W), jnp.float32),
    )(card, seen, flip, valid, t32)
    return out.reshape(B, 1, OUT_W)


# speedup vs baseline: 2.7621x; 1.0003x over previous
import jax
import jax.numpy as jnp
from jax.experimental import pallas as pl

B = 4096
TWO_N = 128
N = 64
OUT_W = TWO_N * N + N + 2
ROWS = 256


def _body(card_ref, seen_ref, flip_ref, valid_ref, t_ref, out_ref):
    R = card_ref.shape[0]
    lane = jax.lax.broadcasted_iota(jnp.int32, (R, 128), 1)
    mod64 = jnp.bitwise_and(lane, 63)
    hi = lane >= 64

    card = card_ref[...]
    seen = seen_ref[...]
    cardm = jnp.where(seen, card, 64)

    for i in range(N):
        c0 = cardm[:, 2 * i : 2 * i + 1]
        c1 = cardm[:, 2 * i + 1 : 2 * i + 2]
        csel = jnp.where(hi, c1, c0)
        out_ref[:, 128 * i : 128 * (i + 1)] = jnp.where(
            csel == mod64, 1.0, 0.0
        )

    f = flip_ref[...]
    fc = jnp.sum(jnp.where(lane == f, card, 0), axis=1, keepdims=True)
    valid = valid_ref[...]
    par = jnp.bitwise_and(t_ref[...], 1)
    flip_val = jnp.where(lane == fc, valid, 0.0)
    par_val = jnp.where((lane - 64) == par, 1.0, 0.0)
    tail = jnp.where(lane < 64, flip_val, par_val)
    out_ref[:, TWO_N * N : OUT_W] = tail[:, : N + 2]


def kernel(card, seen_mask, flipped, flipped_valid, t, W):
    del W
    card = card.astype(jnp.int32)
    seen = seen_mask
    flip = flipped.astype(jnp.int32).reshape(B, 1)
    valid = flipped_valid.astype(jnp.float32).reshape(B, 1)
    t32 = t.astype(jnp.int32).reshape(B, 1)

    grid = (B // ROWS,)
    out = pl.pallas_call(
        _body,
        grid=grid,
        in_specs=[
            pl.BlockSpec((ROWS, TWO_N), lambda i: (i, 0)),
            pl.BlockSpec((ROWS, TWO_N), lambda i: (i, 0)),
            pl.BlockSpec((ROWS, 1), lambda i: (i, 0)),
            pl.BlockSpec((ROWS, 1), lambda i: (i, 0)),
            pl.BlockSpec((ROWS, 1), lambda i: (i, 0)),
        ],
        out_specs=pl.BlockSpec((None, ROWS, OUT_W), lambda i: (0, i, 0)),
        out_shape=jax.ShapeDtypeStruct((1, B, OUT_W), jnp.float32),
    )(card, seen, flip, valid, t32)
    return out.reshape(B, 1, OUT_W)


# speedup vs baseline: 10.2012x; 3.6933x over previous
import jax
import jax.numpy as jnp
from jax.experimental import pallas as pl

B = 4096
TWO_N = 128
N = 64
OUT_W = TWO_N * N + N + 2
FB = 512
P_PER = FB // N
N_MAIN = TWO_N * N // FB


def _body(cardm_ref, card_full_ref, flip_ref, valid_ref, t_ref, out_ref):
    i = pl.program_id(0)

    @pl.when(i < N_MAIN)
    def _main():
        cm = cardm_ref[...]
        sub = jax.lax.broadcasted_iota(jnp.int32, (N, B), 0)
        for j in range(P_PER):
            row = jnp.broadcast_to(cm[j : j + 1, :], (N, B))
            out_ref[N * j : N * (j + 1), :] = jnp.where(row == sub, 1.0, 0.0)

    @pl.when(i == N_MAIN)
    def _tail():
        card = card_full_ref[...]
        flip = flip_ref[...]
        prow = jax.lax.broadcasted_iota(jnp.int32, (TWO_N, B), 0)
        fcv = jnp.where(prow == flip, card, 0)
        fc = jnp.sum(fcv, axis=0, keepdims=True)
        sub = jax.lax.broadcasted_iota(jnp.int32, (N, B), 0)
        valid = valid_ref[...]
        out_ref[0:N, :] = jnp.where(sub == fc, valid, 0.0)
        par = jnp.bitwise_and(t_ref[...], 1)
        sub2 = jax.lax.broadcasted_iota(jnp.int32, (2, B), 0)
        out_ref[N : N + 2, :] = jnp.where(sub2 == par, 1.0, 0.0)


def kernel(card, seen_mask, flipped, flipped_valid, t, W):
    del W
    cardT = card.astype(jnp.int32).T
    cardmT = jnp.where(seen_mask.T, cardT, 64)
    flipT = flipped.astype(jnp.int32).reshape(1, B)
    validT = flipped_valid.astype(jnp.float32).reshape(1, B)
    tT = t.astype(jnp.int32).reshape(1, B)

    grid = (N_MAIN + 1,)
    out = pl.pallas_call(
        _body,
        grid=grid,
        in_specs=[
            pl.BlockSpec((P_PER, B), lambda i: (jnp.minimum(i, N_MAIN - 1), 0)),
            pl.BlockSpec((TWO_N, B), lambda i: (0, 0)),
            pl.BlockSpec((1, B), lambda i: (0, 0)),
            pl.BlockSpec((1, B), lambda i: (0, 0)),
            pl.BlockSpec((1, B), lambda i: (0, 0)),
        ],
        out_specs=pl.BlockSpec((FB, None, B), lambda i: (i, 0, 0)),
        out_shape=jax.ShapeDtypeStruct((OUT_W, 1, B), jnp.float32),
    )(cardmT, cardT, flipT, validT, tT)
    return jnp.transpose(out, (2, 1, 0))
